# Initial kernel scaffold; baseline (speedup 1.0000x reference)
import jax
import jax.numpy as jnp
from jax.experimental import pallas as pl


def kernel(x, e, W1, b1, W2, b2):
    raise NotImplementedError("write your pallas kernel here")


# speedup vs baseline: 16.4219x; 16.4219x over previous
import functools

import jax
import jax.numpy as jnp
from jax import lax
from jax.experimental import pallas as pl
from jax.experimental.pallas import tpu as pltpu
from jax.experimental.pallas import tpu_sc as plsc

N = 10000
E = 320000
D = 128
H = 64

NC = 2
NS = 16
NW = NC * NS
EPT = E // NW
C = 80
NCHUNK = EPT // C
RB = 1000
ZR = 200

_MESH = plsc.VectorSubcoreMesh(core_axis_name="c", subcore_axis_name="s")

_Z16 = functools.partial(jnp.zeros, (16,), jnp.float32)


def _zero1(buf, n16):
    for i in range(n16):
        buf[pl.ds(i * 16, 16)] = _Z16()


def _zero2(buf, rows):
    def zb(r, c):
        for k in range(H // 16):
            buf[r, pl.ds(k * 16, 16)] = _Z16()
        return c
    lax.fori_loop(0, rows, zb, 0)


@functools.partial(
    pl.kernel,
    mesh=_MESH,
    out_type=jax.ShapeDtypeStruct((NC * N,), jnp.float32),
    scratch_types=[
        pltpu.VMEM((C,), jnp.int32),
        pltpu.VMEM((C,), jnp.float32),
        pltpu.VMEM((1008,), jnp.float32),
        pltpu.VMEM_SHARED((N,), jnp.float32),
    ],
)
def _deg_sc(dst_hbm, out_hbm, didx, ones, bounce, acc):
    cid = lax.axis_index("c")
    sid = lax.axis_index("s")
    wid = cid * NS + sid

    for i in range(C // 16):
        ones[pl.ds(i * 16, 16)] = jnp.full((16,), 1.0, jnp.float32)

    @pl.when(sid < N // RB)
    def _():
        _zero1(bounce, 63)
        pltpu.sync_copy(bounce.at[pl.ds(0, RB)], acc.at[pl.ds(sid * RB, RB)])

    plsc.subcore_barrier()

    base0 = wid * EPT

    def body(j, carry):
        pltpu.sync_copy(dst_hbm.at[pl.ds(base0 + j * C, C)], didx)
        pltpu.sync_copy(ones, acc.at[didx], add=True)
        return carry

    lax.fori_loop(0, NCHUNK, body, 0)

    plsc.subcore_barrier()

    @pl.when(sid < N // RB)
    def _():
        pltpu.sync_copy(acc.at[pl.ds(sid * RB, RB)], bounce.at[pl.ds(0, RB)])
        pltpu.sync_copy(bounce.at[pl.ds(0, RB)],
                        out_hbm.at[pl.ds(cid * N + sid * RB, RB)])


@functools.partial(
    pl.kernel,
    mesh=_MESH,
    compiler_params=pltpu.CompilerParams(use_tc_tiling_on_sc=False),
    out_type=jax.ShapeDtypeStruct((NC, N, H), jnp.float32),
    scratch_types=[
        pltpu.VMEM((C,), jnp.int32),
        pltpu.VMEM((C,), jnp.int32),
        pltpu.VMEM((C, H), jnp.float32),
        pltpu.VMEM((ZR, H), jnp.float32),
        pltpu.VMEM_SHARED((N, H), jnp.float32),
        pltpu.SemaphoreType.DMA,
    ],
)
def _prop64_sc(y_hbm, src_hbm, dst_hbm, out_hbm,
               sidx, didx, rows, bounce, acc, sem):
    cid = lax.axis_index("c")
    sid = lax.axis_index("s")
    wid = cid * NS + sid

    @pl.when(sid < N // RB)
    def _():
        _zero2(bounce, ZR)
        for k in range(RB // ZR):
            pltpu.sync_copy(bounce, acc.at[pl.ds(sid * RB + k * ZR, ZR)])

    plsc.subcore_barrier()

    base0 = wid * EPT

    def body(j, carry):
        base = base0 + j * C
        pltpu.sync_copy(src_hbm.at[pl.ds(base, C)], sidx)
        pltpu.sync_copy(dst_hbm.at[pl.ds(base, C)], didx)
        pltpu.async_copy(y_hbm.at[sidx], rows, sem).wait()
        pltpu.sync_copy(rows, acc.at[didx], add=True)
        return carry

    lax.fori_loop(0, NCHUNK, body, 0)

    plsc.subcore_barrier()

    @pl.when(sid < N // RB)
    def _():
        for k in range(RB // ZR):
            r0 = sid * RB + k * ZR
            pltpu.sync_copy(acc.at[pl.ds(r0, ZR)], bounce)
            pltpu.sync_copy(bounce, out_hbm.at[cid, pl.ds(r0, ZR)])


@functools.partial(
    pl.kernel,
    mesh=_MESH,
    out_type=jax.ShapeDtypeStruct((NC * N,), jnp.float32),
    scratch_types=[
        pltpu.VMEM((C,), jnp.int32),
        pltpu.VMEM((C,), jnp.int32),
        pltpu.VMEM((C,), jnp.float32),
        pltpu.VMEM((1008,), jnp.float32),
        pltpu.VMEM_SHARED((N,), jnp.float32),
        pltpu.SemaphoreType.DMA,
    ],
)
def _prop1_sc(y_hbm, src_hbm, dst_hbm, out_hbm,
              sidx, didx, vals, bounce, acc, sem):
    cid = lax.axis_index("c")
    sid = lax.axis_index("s")
    wid = cid * NS + sid

    @pl.when(sid < N // RB)
    def _():
        _zero1(bounce, 63)
        pltpu.sync_copy(bounce.at[pl.ds(0, RB)], acc.at[pl.ds(sid * RB, RB)])

    plsc.subcore_barrier()

    base0 = wid * EPT

    def body(j, carry):
        base = base0 + j * C
        pltpu.sync_copy(src_hbm.at[pl.ds(base, C)], sidx)
        pltpu.sync_copy(dst_hbm.at[pl.ds(base, C)], didx)
        pltpu.async_copy(y_hbm.at[sidx], vals, sem).wait()
        pltpu.sync_copy(vals, acc.at[didx], add=True)
        return carry

    lax.fori_loop(0, NCHUNK, body, 0)

    plsc.subcore_barrier()

    @pl.when(sid < N // RB)
    def _():
        pltpu.sync_copy(acc.at[pl.ds(sid * RB, RB)], bounce.at[pl.ds(0, RB)])
        pltpu.sync_copy(bounce.at[pl.ds(0, RB)],
                        out_hbm.at[pl.ds(cid * N + sid * RB, RB)])


_GB = 1000


def _mm_body(x_ref, w_ref, o_ref):
    o_ref[...] = jnp.dot(x_ref[...], w_ref[...],
                         preferred_element_type=jnp.float32)


def _mm(x, W1):
    return pl.pallas_call(
        _mm_body,
        grid=(N // _GB,),
        in_specs=[pl.BlockSpec((_GB, D), lambda i: (i, 0)),
                  pl.BlockSpec((D, H), lambda i: (0, 0))],
        out_specs=pl.BlockSpec((_GB, H), lambda i: (i, 0)),
        out_shape=jax.ShapeDtypeStruct((N, H), jnp.float32),
    )(x, W1)


def _prep_body(degp_ref, xw_ref, dinv_ref, y1_ref):
    deg = degp_ref[0] + degp_ref[1] + 1.0
    dinv = lax.rsqrt(deg)
    dinv_ref[...] = dinv
    y1_ref[...] = xw_ref[...] * dinv


def _prep(degp, xw):
    return pl.pallas_call(
        _prep_body,
        grid=(N // _GB,),
        in_specs=[pl.BlockSpec((NC, _GB, 1), lambda i: (0, i, 0)),
                  pl.BlockSpec((_GB, H), lambda i: (i, 0))],
        out_specs=[pl.BlockSpec((_GB, 1), lambda i: (i, 0)),
                   pl.BlockSpec((_GB, H), lambda i: (i, 0))],
        out_shape=[jax.ShapeDtypeStruct((N, 1), jnp.float32),
                   jax.ShapeDtypeStruct((N, H), jnp.float32)],
    )(degp, xw)


def _l2_body(accp_ref, y1_ref, dinv_ref, w2_ref, b1_ref, y2_ref):
    a = accp_ref[0] + accp_ref[1] + y1_ref[...]
    h = jnp.maximum(a * dinv_ref[...] + b1_ref[...], 0.0)
    hw = jnp.dot(h, w2_ref[...], preferred_element_type=jnp.float32)
    y2_ref[...] = hw * dinv_ref[...]


def _l2(accp, y1, dinv, W2, b1_row):
    return pl.pallas_call(
        _l2_body,
        grid=(N // _GB,),
        in_specs=[pl.BlockSpec((NC, _GB, H), lambda i: (0, i, 0)),
                  pl.BlockSpec((_GB, H), lambda i: (i, 0)),
                  pl.BlockSpec((_GB, 1), lambda i: (i, 0)),
                  pl.BlockSpec((H, 1), lambda i: (0, 0)),
                  pl.BlockSpec((1, H), lambda i: (0, 0))],
        out_specs=pl.BlockSpec((_GB, 1), lambda i: (i, 0)),
        out_shape=jax.ShapeDtypeStruct((N, 1), jnp.float32),
    )(accp, y1, dinv, W2, b1_row)


def _fin_body(accp_ref, y2_ref, dinv_ref, b2_ref, out_ref):
    a = accp_ref[0] + accp_ref[1]
    out_ref[...] = (a + y2_ref[...]) * dinv_ref[...] + b2_ref[...]


def _fin(accp2, y2, dinv, b2_row):
    return pl.pallas_call(
        _fin_body,
        grid=(N // _GB,),
        in_specs=[pl.BlockSpec((NC, _GB, 1), lambda i: (0, i, 0)),
                  pl.BlockSpec((_GB, 1), lambda i: (i, 0)),
                  pl.BlockSpec((_GB, 1), lambda i: (i, 0)),
                  pl.BlockSpec((1, 1), lambda i: (0, 0))],
        out_specs=pl.BlockSpec((_GB, 1), lambda i: (i, 0)),
        out_shape=jax.ShapeDtypeStruct((N, 1), jnp.float32),
    )(accp2, y2, dinv, b2_row)


def kernel(x, e, W1, b1, W2, b2):
    src = e[0]
    dst = e[1]

    degp = jnp.reshape(_deg_sc(dst), (NC, N, 1))
    xw = _mm(x, W1)
    dinv, y1 = _prep(degp, xw)
    accp = _prop64_sc(y1, src, dst)
    y2 = _l2(accp, y1, dinv, W2, jnp.reshape(b1, (1, H)))
    acc2 = _prop1_sc(jnp.reshape(y2, (N,)), src, dst)
    out = _fin(jnp.reshape(acc2, (NC, N, 1)), y2, dinv, jnp.reshape(b2, (1, 1)))
    return out


# speedup vs baseline: 43.5712x; 2.6532x over previous
import functools

import jax
import jax.numpy as jnp
from jax import lax
from jax.experimental import pallas as pl
from jax.experimental.pallas import tpu as pltpu
from jax.experimental.pallas import tpu_sc as plsc

N = 10000
E = 320000
D = 128
H = 64

NC = 2
NS = 16
NW = NC * NS
EPT = E // NW
CP = 125
NCH = EPT // CP
NBUF = 5
CD = 80
NCHD = EPT // CD
GD = 5
RB = 1000
ZR = 200

_MESH = plsc.VectorSubcoreMesh(core_axis_name="c", subcore_axis_name="s")
_NOTC = pltpu.CompilerParams(use_tc_tiling_on_sc=False)

_Z16 = functools.partial(jnp.zeros, (16,), jnp.float32)


def _zero1(buf, n16):
    for i in range(n16):
        buf[pl.ds(i * 16, 16)] = _Z16()


def _zero2(buf, rows):
    def zb(r, c):
        for k in range(H // 16):
            buf[r, pl.ds(k * 16, 16)] = _Z16()
        return c
    lax.fori_loop(0, rows, zb, 0)


@functools.partial(
    pl.kernel,
    mesh=_MESH,
    compiler_params=_NOTC,
    out_type=jax.ShapeDtypeStruct((NC * N,), jnp.float32),
    scratch_types=[
        pltpu.VMEM((NCHD, CD), jnp.int32),
        pltpu.VMEM((CD,), jnp.float32),
        pltpu.VMEM((1008,), jnp.float32),
        pltpu.VMEM_SHARED((N,), jnp.float32),
        pltpu.SemaphoreType.DMA,
    ],
)
def _deg_sc(dst_hbm, out_hbm, didx, ones, bounce, acc, sem):
    cid = lax.axis_index("c")
    sid = lax.axis_index("s")
    wid = cid * NS + sid

    pltpu.sync_copy(dst_hbm.at[wid], didx)
    for i in range(CD // 16):
        ones[pl.ds(i * 16, 16)] = jnp.full((16,), 1.0, jnp.float32)

    @pl.when(sid < N // RB)
    def _():
        _zero1(bounce, 63)
        pltpu.sync_copy(bounce.at[pl.ds(0, RB)], acc.at[pl.ds(sid * RB, RB)])

    plsc.subcore_barrier()

    def body(g, carry):
        descs = [
            pltpu.async_copy(ones, acc.at[didx.at[g * GD + b]], sem, add=True)
            for b in range(GD)
        ]
        for d in descs:
            d.wait()
        return carry

    lax.fori_loop(0, NCHD // GD, body, 0)

    plsc.subcore_barrier()

    @pl.when(sid < N // RB)
    def _():
        pltpu.sync_copy(acc.at[pl.ds(sid * RB, RB)], bounce.at[pl.ds(0, RB)])
        pltpu.sync_copy(bounce.at[pl.ds(0, RB)],
                        out_hbm.at[pl.ds(cid * N + sid * RB, RB)])


@functools.partial(
    pl.kernel,
    mesh=_MESH,
    compiler_params=_NOTC,
    out_type=jax.ShapeDtypeStruct((NC, N, H), jnp.float32),
    scratch_types=(
        [pltpu.VMEM((NCH, CP), jnp.int32),
         pltpu.VMEM((NCH, CP), jnp.int32)]
        + [pltpu.VMEM((CP, H), jnp.float32) for _ in range(NBUF)]
        + [pltpu.VMEM((ZR, H), jnp.float32),
           pltpu.VMEM_SHARED((N, H), jnp.float32)]
        + [pltpu.SemaphoreType.DMA for _ in range(2 * NBUF)]
    ),
)
def _prop64_sc(y_hbm, src_hbm, dst_hbm, out_hbm, sidx, didx, *rest):
    rows = rest[:NBUF]
    bounce = rest[NBUF]
    acc = rest[NBUF + 1]
    sem_g = rest[NBUF + 2:2 * NBUF + 2]
    sem_s = rest[2 * NBUF + 2:]
    cid = lax.axis_index("c")
    sid = lax.axis_index("s")
    wid = cid * NS + sid

    pltpu.sync_copy(src_hbm.at[wid], sidx)
    pltpu.sync_copy(dst_hbm.at[wid], didx)

    @pl.when(sid < N // RB)
    def _():
        _zero2(bounce, ZR)
        for k in range(RB // ZR):
            pltpu.sync_copy(bounce, acc.at[pl.ds(sid * RB + k * ZR, ZR)])

    plsc.subcore_barrier()

    def body(g, carry):
        gd = [
            pltpu.async_copy(y_hbm.at[sidx.at[g * NBUF + b]], rows[b],
                             sem_g[b])
            for b in range(NBUF)
        ]
        sd = []
        for b in range(NBUF):
            gd[b].wait()
            sd.append(pltpu.async_copy(rows[b],
                                       acc.at[didx.at[g * NBUF + b]],
                                       sem_s[b], add=True))
        for d in sd:
            d.wait()
        return carry

    lax.fori_loop(0, NCH // NBUF, body, 0)

    plsc.subcore_barrier()

    @pl.when(sid < N // RB)
    def _():
        for k in range(RB // ZR):
            r0 = sid * RB + k * ZR
            pltpu.sync_copy(acc.at[pl.ds(r0, ZR)], bounce)
            pltpu.sync_copy(bounce, out_hbm.at[cid, pl.ds(r0, ZR)])


@functools.partial(
    pl.kernel,
    mesh=_MESH,
    compiler_params=_NOTC,
    out_type=jax.ShapeDtypeStruct((NC * N,), jnp.float32),
    scratch_types=(
        [pltpu.VMEM((NCH, CP), jnp.int32),
         pltpu.VMEM((NCH, CP), jnp.int32)]
        + [pltpu.VMEM((CP,), jnp.float32) for _ in range(NBUF)]
        + [pltpu.VMEM((1008,), jnp.float32),
           pltpu.VMEM_SHARED((N,), jnp.float32)]
        + [pltpu.SemaphoreType.DMA for _ in range(2 * NBUF)]
    ),
)
def _prop1_sc(y_hbm, src_hbm, dst_hbm, out_hbm, sidx, didx, *rest):
    vals = rest[:NBUF]
    bounce = rest[NBUF]
    acc = rest[NBUF + 1]
    sem_g = rest[NBUF + 2:2 * NBUF + 2]
    sem_s = rest[2 * NBUF + 2:]
    cid = lax.axis_index("c")
    sid = lax.axis_index("s")
    wid = cid * NS + sid

    pltpu.sync_copy(src_hbm.at[wid], sidx)
    pltpu.sync_copy(dst_hbm.at[wid], didx)

    @pl.when(sid < N // RB)
    def _():
        _zero1(bounce, 63)
        pltpu.sync_copy(bounce.at[pl.ds(0, RB)], acc.at[pl.ds(sid * RB, RB)])

    plsc.subcore_barrier()

    def body(g, carry):
        gd = [
            pltpu.async_copy(y_hbm.at[sidx.at[g * NBUF + b]], vals[b],
                             sem_g[b])
            for b in range(NBUF)
        ]
        sd = []
        for b in range(NBUF):
            gd[b].wait()
            sd.append(pltpu.async_copy(vals[b],
                                       acc.at[didx.at[g * NBUF + b]],
                                       sem_s[b], add=True))
        for d in sd:
            d.wait()
        return carry

    lax.fori_loop(0, NCH // NBUF, body, 0)

    plsc.subcore_barrier()

    @pl.when(sid < N // RB)
    def _():
        pltpu.sync_copy(acc.at[pl.ds(sid * RB, RB)], bounce.at[pl.ds(0, RB)])
        pltpu.sync_copy(bounce.at[pl.ds(0, RB)],
                        out_hbm.at[pl.ds(cid * N + sid * RB, RB)])


_GB = 1000


def _mm_body(x_ref, w_ref, o_ref):
    o_ref[...] = jnp.dot(x_ref[...], w_ref[...],
                         preferred_element_type=jnp.float32)


def _mm(x, W1):
    return pl.pallas_call(
        _mm_body,
        grid=(N // _GB,),
        in_specs=[pl.BlockSpec((_GB, D), lambda i: (i, 0)),
                  pl.BlockSpec((D, H), lambda i: (0, 0))],
        out_specs=pl.BlockSpec((_GB, H), lambda i: (i, 0)),
        out_shape=jax.ShapeDtypeStruct((N, H), jnp.float32),
    )(x, W1)


def _prep_body(degp_ref, xw_ref, dinv_ref, y1_ref):
    deg = degp_ref[0] + degp_ref[1] + 1.0
    dinv = lax.rsqrt(deg)
    dinv_ref[...] = dinv
    y1_ref[...] = xw_ref[...] * dinv


def _prep(degp, xw):
    return pl.pallas_call(
        _prep_body,
        grid=(N // _GB,),
        in_specs=[pl.BlockSpec((NC, _GB, 1), lambda i: (0, i, 0)),
                  pl.BlockSpec((_GB, H), lambda i: (i, 0))],
        out_specs=[pl.BlockSpec((_GB, 1), lambda i: (i, 0)),
                   pl.BlockSpec((_GB, H), lambda i: (i, 0))],
        out_shape=[jax.ShapeDtypeStruct((N, 1), jnp.float32),
                   jax.ShapeDtypeStruct((N, H), jnp.float32)],
    )(degp, xw)


def _l2_body(accp_ref, y1_ref, dinv_ref, w2_ref, b1_ref, y2_ref):
    a = accp_ref[0] + accp_ref[1] + y1_ref[...]
    h = jnp.maximum(a * dinv_ref[...] + b1_ref[...], 0.0)
    hw = jnp.dot(h, w2_ref[...], preferred_element_type=jnp.float32)
    y2_ref[...] = hw * dinv_ref[...]


def _l2(accp, y1, dinv, W2, b1_row):
    return pl.pallas_call(
        _l2_body,
        grid=(N // _GB,),
        in_specs=[pl.BlockSpec((NC, _GB, H), lambda i: (0, i, 0)),
                  pl.BlockSpec((_GB, H), lambda i: (i, 0)),
                  pl.BlockSpec((_GB, 1), lambda i: (i, 0)),
                  pl.BlockSpec((H, 1), lambda i: (0, 0)),
                  pl.BlockSpec((1, H), lambda i: (0, 0))],
        out_specs=pl.BlockSpec((_GB, 1), lambda i: (i, 0)),
        out_shape=jax.ShapeDtypeStruct((N, 1), jnp.float32),
    )(accp, y1, dinv, W2, b1_row)


def _fin_body(accp_ref, y2_ref, dinv_ref, b2_ref, out_ref):
    a = accp_ref[0] + accp_ref[1]
    out_ref[...] = (a + y2_ref[...]) * dinv_ref[...] + b2_ref[...]


def _fin(accp2, y2, dinv, b2_row):
    return pl.pallas_call(
        _fin_body,
        grid=(N // _GB,),
        in_specs=[pl.BlockSpec((NC, _GB, 1), lambda i: (0, i, 0)),
                  pl.BlockSpec((_GB, 1), lambda i: (i, 0)),
                  pl.BlockSpec((_GB, 1), lambda i: (i, 0)),
                  pl.BlockSpec((1, 1), lambda i: (0, 0))],
        out_specs=pl.BlockSpec((_GB, 1), lambda i: (i, 0)),
        out_shape=jax.ShapeDtypeStruct((N, 1), jnp.float32),
    )(accp2, y2, dinv, b2_row)


def kernel(x, e, W1, b1, W2, b2):
    src3 = jnp.reshape(e[0], (NW, NCH, CP))
    dst3 = jnp.reshape(e[1], (NW, NCH, CP))
    dst3d = jnp.reshape(e[1], (NW, NCHD, CD))

    degp = jnp.reshape(_deg_sc(dst3d), (NC, N, 1))
    xw = _mm(x, W1)
    dinv, y1 = _prep(degp, xw)
    accp = _prop64_sc(y1, src3, dst3)
    y2 = _l2(accp, y1, dinv, W2, jnp.reshape(b1, (1, H)))
    acc2 = _prop1_sc(jnp.reshape(y2, (N,)), src3, dst3)
    out = _fin(jnp.reshape(acc2, (NC, N, 1)), y2, dinv, jnp.reshape(b2, (1, 1)))
    return out


# speedup vs baseline: 60.4920x; 1.3883x over previous
import functools

import jax
import jax.numpy as jnp
from jax import lax
from jax.experimental import pallas as pl
from jax.experimental.pallas import tpu as pltpu
from jax.experimental.pallas import tpu_sc as plsc

N = 10000
E = 320000
D = 128
H = 64

NC = 2
NS = 16
NW = NC * NS
EPT = E // NW
CP = 125
NCH = EPT // CP
NBUF = 8
GD = 5
RB = 1000
ZR = 200

_MESH = plsc.VectorSubcoreMesh(core_axis_name="c", subcore_axis_name="s")
_NOTC = pltpu.CompilerParams(use_tc_tiling_on_sc=False)

_Z16 = functools.partial(jnp.zeros, (16,), jnp.float32)


def _zero1(buf, n16):
    for i in range(n16):
        buf[pl.ds(i * 16, 16)] = _Z16()


def _zero2(buf, rows):
    def zb(r, c):
        for k in range(H // 16):
            buf[r, pl.ds(k * 16, 16)] = _Z16()
        return c
    lax.fori_loop(0, rows, zb, 0)


@functools.partial(
    pl.kernel,
    mesh=_MESH,
    compiler_params=_NOTC,
    out_type=jax.ShapeDtypeStruct((NC * N,), jnp.float32),
    scratch_types=[
        pltpu.VMEM((NCH, CP), jnp.int32),
        pltpu.VMEM((128,), jnp.float32),
        pltpu.VMEM((1008,), jnp.float32),
        pltpu.VMEM_SHARED((N,), jnp.float32),
        pltpu.SemaphoreType.DMA,
    ],
)
def _deg_sc(e3_hbm, out_hbm, didx, ones, bounce, acc, sem):
    cid = lax.axis_index("c")
    sid = lax.axis_index("s")
    wid = cid * NS + sid

    pltpu.sync_copy(e3_hbm.at[1, wid], didx)
    for i in range(8):
        ones[pl.ds(i * 16, 16)] = jnp.full((16,), 1.0, jnp.float32)

    @pl.when(sid < N // RB)
    def _():
        _zero1(bounce, 63)
        pltpu.sync_copy(bounce.at[pl.ds(0, RB)], acc.at[pl.ds(sid * RB, RB)])

    plsc.subcore_barrier()

    def body(g, carry):
        descs = [
            pltpu.async_copy(ones.at[pl.ds(0, CP)],
                             acc.at[didx.at[g * GD + b]], sem, add=True)
            for b in range(GD)
        ]
        for d in descs:
            d.wait()
        return carry

    lax.fori_loop(0, NCH // GD, body, 0)

    plsc.subcore_barrier()

    @pl.when(sid < N // RB)
    def _():
        pltpu.sync_copy(acc.at[pl.ds(sid * RB, RB)], bounce.at[pl.ds(0, RB)])
        pltpu.sync_copy(bounce.at[pl.ds(0, RB)],
                        out_hbm.at[pl.ds(cid * N + sid * RB, RB)])


@functools.partial(
    pl.kernel,
    mesh=_MESH,
    compiler_params=_NOTC,
    out_type=jax.ShapeDtypeStruct((NC, N, H), jnp.float32),
    scratch_types=(
        [pltpu.VMEM((NCH, CP), jnp.int32),
         pltpu.VMEM((NCH, CP), jnp.int32)]
        + [pltpu.VMEM((CP, H), jnp.float32) for _ in range(NBUF)]
        + [pltpu.VMEM_SHARED((N, H), jnp.float32)]
        + [pltpu.SemaphoreType.DMA for _ in range(2 * NBUF)]
    ),
)
def _prop64_sc(y_hbm, e3_hbm, out_hbm, sidx, didx, *rest):
    rows = rest[:NBUF]
    acc = rest[NBUF]
    sem_g = rest[NBUF + 1:2 * NBUF + 1]
    sem_s = rest[2 * NBUF + 1:]
    cid = lax.axis_index("c")
    sid = lax.axis_index("s")
    wid = cid * NS + sid
    nrt = N // NS

    pltpu.sync_copy(e3_hbm.at[0, wid], sidx)
    pltpu.sync_copy(e3_hbm.at[1, wid], didx)

    _zero2(rows[0], CP)
    for k in range(nrt // CP):
        pltpu.sync_copy(rows[0], acc.at[pl.ds(sid * nrt + k * CP, CP)])

    plsc.subcore_barrier()

    def body(g, carry):
        gd = [
            pltpu.async_copy(y_hbm.at[sidx.at[g * NBUF + b]], rows[b],
                             sem_g[b])
            for b in range(NBUF)
        ]
        sd = []
        for b in range(NBUF):
            gd[b].wait()
            sd.append(pltpu.async_copy(rows[b],
                                       acc.at[didx.at[g * NBUF + b]],
                                       sem_s[b], add=True))
        for d in sd:
            d.wait()
        return carry

    lax.fori_loop(0, NCH // NBUF, body, 0)

    plsc.subcore_barrier()

    wo = []
    for k in range(nrt // CP):
        r0 = sid * nrt + k * CP
        wo.append(pltpu.async_copy(acc.at[pl.ds(r0, CP)], rows[k], sem_g[k]))
    wo2 = []
    for k in range(nrt // CP):
        r0 = sid * nrt + k * CP
        wo[k].wait()
        wo2.append(pltpu.async_copy(rows[k], out_hbm.at[cid, pl.ds(r0, CP)],
                                    sem_s[k]))
    for d in wo2:
        d.wait()


@functools.partial(
    pl.kernel,
    mesh=_MESH,
    compiler_params=_NOTC,
    out_type=jax.ShapeDtypeStruct((NC * N,), jnp.float32),
    scratch_types=(
        [pltpu.VMEM((NCH, CP), jnp.int32),
         pltpu.VMEM((NCH, CP), jnp.int32)]
        + [pltpu.VMEM((CP,), jnp.float32) for _ in range(NBUF)]
        + [pltpu.VMEM((1008,), jnp.float32),
           pltpu.VMEM_SHARED((N,), jnp.float32)]
        + [pltpu.SemaphoreType.DMA for _ in range(2 * NBUF)]
    ),
)
def _prop1_sc(y_hbm, e3_hbm, out_hbm, sidx, didx, *rest):
    vals = rest[:NBUF]
    bounce = rest[NBUF]
    acc = rest[NBUF + 1]
    sem_g = rest[NBUF + 2:2 * NBUF + 2]
    sem_s = rest[2 * NBUF + 2:]
    cid = lax.axis_index("c")
    sid = lax.axis_index("s")
    wid = cid * NS + sid

    pltpu.sync_copy(e3_hbm.at[0, wid], sidx)
    pltpu.sync_copy(e3_hbm.at[1, wid], didx)

    @pl.when(sid < N // RB)
    def _():
        _zero1(bounce, 63)
        pltpu.sync_copy(bounce.at[pl.ds(0, RB)], acc.at[pl.ds(sid * RB, RB)])

    plsc.subcore_barrier()

    def body(g, carry):
        gd = [
            pltpu.async_copy(y_hbm.at[sidx.at[g * NBUF + b]], vals[b],
                             sem_g[b])
            for b in range(NBUF)
        ]
        sd = []
        for b in range(NBUF):
            gd[b].wait()
            sd.append(pltpu.async_copy(vals[b],
                                       acc.at[didx.at[g * NBUF + b]],
                                       sem_s[b], add=True))
        for d in sd:
            d.wait()
        return carry

    lax.fori_loop(0, NCH // NBUF, body, 0)

    plsc.subcore_barrier()

    @pl.when(sid < N // RB)
    def _():
        pltpu.sync_copy(acc.at[pl.ds(sid * RB, RB)], bounce.at[pl.ds(0, RB)])
        pltpu.sync_copy(bounce.at[pl.ds(0, RB)],
                        out_hbm.at[pl.ds(cid * N + sid * RB, RB)])


def _mm_body(x_ref, w_ref, o_ref):
    o_ref[...] = jnp.dot(x_ref[...], w_ref[...],
                         preferred_element_type=jnp.float32)


def _mm(x, W1):
    return pl.pallas_call(
        _mm_body,
        out_shape=jax.ShapeDtypeStruct((N, H), jnp.float32),
    )(x, W1)


def _prep_body(degp_ref, xw_ref, dinv_ref, y1_ref):
    deg = degp_ref[pl.ds(0, N)] + degp_ref[pl.ds(N, N)] + 1.0
    dinv = lax.rsqrt(deg)
    dinv_ref[...] = dinv
    y1_ref[...] = xw_ref[...] * dinv[:, None]


def _prep(degp, xw):
    return pl.pallas_call(
        _prep_body,
        out_shape=[jax.ShapeDtypeStruct((N,), jnp.float32),
                   jax.ShapeDtypeStruct((N, H), jnp.float32)],
    )(degp, xw)


def _l2_body(accp_ref, y1_ref, dinv_ref, w2_ref, b1_ref, y2_ref):
    a = accp_ref[0] + accp_ref[1] + y1_ref[...]
    h = jnp.maximum(a * dinv_ref[...][:, None] + b1_ref[...], 0.0)
    hw = jnp.dot(h, w2_ref[...], preferred_element_type=jnp.float32)
    y2_ref[...] = hw[:, 0] * dinv_ref[...]


def _l2(accp, y1, dinv, W2, b1_row):
    return pl.pallas_call(
        _l2_body,
        out_shape=jax.ShapeDtypeStruct((N,), jnp.float32),
    )(accp, y1, dinv, W2, b1_row)


def _fin_body(acc2_ref, y2_ref, dinv_ref, b2_ref, out_ref):
    a = acc2_ref[pl.ds(0, N)] + acc2_ref[pl.ds(N, N)]
    out_ref[...] = ((a + y2_ref[...]) * dinv_ref[...] + b2_ref[0])[:, None]


def _fin(acc2, y2, dinv, b2):
    return pl.pallas_call(
        _fin_body,
        out_shape=jax.ShapeDtypeStruct((N, 1), jnp.float32),
    )(acc2, y2, dinv, b2)


def kernel(x, e, W1, b1, W2, b2):
    e3 = jnp.reshape(e, (2, NW, NCH, CP))

    degp = _deg_sc(e3)
    xw = _mm(x, W1)
    dinv, y1 = _prep(degp, xw)
    accp = _prop64_sc(y1, e3)
    y2 = _l2(accp, y1, dinv, W2, jnp.reshape(b1, (1, H)))
    acc2 = _prop1_sc(y2, e3)
    out = _fin(acc2, y2, dinv, b2)
    return out


# speedup vs baseline: 71.1336x; 1.1759x over previous
import functools

import jax
import jax.numpy as jnp
from jax import lax
from jax.experimental import pallas as pl
from jax.experimental.pallas import tpu as pltpu
from jax.experimental.pallas import tpu_sc as plsc

N = 10000
E = 320000
D = 128
H = 64

NC = 2
NS = 16
NW = NC * NS
EPT = E // NW
CP = 125
NCH = EPT // CP
NBUF = 8
NV = EPT // 16
REG = 640

_MESH = plsc.VectorSubcoreMesh(core_axis_name="c", subcore_axis_name="s")
_NOTC = pltpu.CompilerParams(use_tc_tiling_on_sc=False,
                             needs_layout_passes=False)

_Z16 = functools.partial(jnp.zeros, (16,), jnp.float32)


def _zero2(buf, rows):
    def zb(r, c):
        for k in range(H // 16):
            buf[r, pl.ds(k * 16, 16)] = _Z16()
        return c
    lax.fori_loop(0, rows, zb, 0)


def _zero1_loop(buf, nvec):
    def zb(i, c):
        buf[pl.ds(i * 16, 16)] = _Z16()
        return c
    lax.fori_loop(0, nvec, zb, 0)


def _hist_reduce_out(stg, hbuf, obuf, out_hbm, cid, sid):
    def _region(r0, nv16):
        pltpu.sync_copy(stg.at[:, pl.ds(r0, nv16 * 16)],
                        hbuf.at[:, pl.ds(0, nv16 * 16)])

        def red(i, c):
            s = hbuf[0, pl.ds(i * 16, 16)]
            for k in range(1, NS):
                s = s + hbuf[k, pl.ds(i * 16, 16)]
            obuf[pl.ds(i * 16, 16)] = s
            return c
        lax.fori_loop(0, nv16, red, 0)
        pltpu.sync_copy(obuf.at[pl.ds(0, nv16 * 16)],
                        out_hbm.at[pl.ds(cid * N + r0, nv16 * 16)])

    @pl.when(sid < NS - 1)
    def _():
        _region(sid * REG, REG // 16)

    @pl.when(sid == NS - 1)
    def _():
        _region((NS - 1) * REG, (N - (NS - 1) * REG) // 16)


@functools.partial(
    pl.kernel,
    mesh=_MESH,
    compiler_params=_NOTC,
    out_type=jax.ShapeDtypeStruct((NC * N,), jnp.float32),
    scratch_types=[
        pltpu.VMEM((NV, 16), jnp.int32),
        pltpu.VMEM((N,), jnp.float32),
        pltpu.VMEM((NS, REG), jnp.float32),
        pltpu.VMEM((REG,), jnp.float32),
        pltpu.VMEM_SHARED((NS, N), jnp.float32),
    ],
)
def _deg_sc(e4_hbm, out_hbm, didx, hist, hbuf, obuf, stg):
    cid = lax.axis_index("c")
    sid = lax.axis_index("s")
    wid = cid * NS + sid

    pltpu.sync_copy(e4_hbm.at[1, wid], didx)
    _zero1_loop(hist, N // 16)

    ones = jnp.full((16,), 1.0, jnp.float32)

    def body(j, c):
        di = didx[j]
        plsc.addupdate_scatter(hist, [di], ones)
        return c

    lax.fori_loop(0, NV, body, 0)

    pltpu.sync_copy(hist, stg.at[sid])
    plsc.subcore_barrier()
    _hist_reduce_out(stg, hbuf, obuf, out_hbm, cid, sid)


@functools.partial(
    pl.kernel,
    mesh=_MESH,
    compiler_params=_NOTC,
    out_type=jax.ShapeDtypeStruct((NC, N, H), jnp.float32),
    scratch_types=(
        [pltpu.VMEM((NCH, CP), jnp.int32),
         pltpu.VMEM((NCH, CP), jnp.int32)]
        + [pltpu.VMEM((CP, H), jnp.float32) for _ in range(NBUF)]
        + [pltpu.VMEM_SHARED((N, H), jnp.float32)]
        + [pltpu.SemaphoreType.DMA for _ in range(2 * NBUF)]
    ),
)
def _prop64_sc(y_hbm, e3_hbm, out_hbm, sidx, didx, *rest):
    rows = rest[:NBUF]
    acc = rest[NBUF]
    sem_g = rest[NBUF + 1:2 * NBUF + 1]
    sem_s = rest[2 * NBUF + 1:]
    cid = lax.axis_index("c")
    sid = lax.axis_index("s")
    wid = cid * NS + sid
    nrt = N // NS

    pltpu.sync_copy(e3_hbm.at[0, wid], sidx)
    pltpu.sync_copy(e3_hbm.at[1, wid], didx)

    _zero2(rows[0], CP)
    for k in range(nrt // CP):
        pltpu.sync_copy(rows[0], acc.at[pl.ds(sid * nrt + k * CP, CP)])

    plsc.subcore_barrier()

    def body(g, carry):
        gd = []
        for b in range(NBUF):
            @pl.when(g > 0)
            def _(b=b):
                pltpu.make_async_copy(rows[b], acc.at[didx.at[0]],
                                      sem_s[b]).wait()
            gd.append(pltpu.async_copy(y_hbm.at[sidx.at[g * NBUF + b]],
                                       rows[b], sem_g[b]))
        for b in range(NBUF):
            gd[b].wait()
            pltpu.async_copy(rows[b], acc.at[didx.at[g * NBUF + b]],
                             sem_s[b], add=True)
        return carry

    lax.fori_loop(0, NCH // NBUF, body, 0)
    for b in range(NBUF):
        pltpu.make_async_copy(rows[b], acc.at[didx.at[0]], sem_s[b]).wait()

    plsc.subcore_barrier()

    wo = []
    for k in range(nrt // CP):
        r0 = sid * nrt + k * CP
        wo.append(pltpu.async_copy(acc.at[pl.ds(r0, CP)], rows[k], sem_g[k]))
    wo2 = []
    for k in range(nrt // CP):
        r0 = sid * nrt + k * CP
        wo[k].wait()
        wo2.append(pltpu.async_copy(rows[k], out_hbm.at[cid, pl.ds(r0, CP)],
                                    sem_s[k]))
    for d in wo2:
        d.wait()


@functools.partial(
    pl.kernel,
    mesh=_MESH,
    compiler_params=_NOTC,
    out_type=jax.ShapeDtypeStruct((NC * N,), jnp.float32),
    scratch_types=[
        pltpu.VMEM((NV, 16), jnp.int32),
        pltpu.VMEM((NV, 16), jnp.int32),
        pltpu.VMEM((N,), jnp.float32),
        pltpu.VMEM((N,), jnp.float32),
        pltpu.VMEM((NS, REG), jnp.float32),
        pltpu.VMEM((REG,), jnp.float32),
        pltpu.VMEM_SHARED((NS, N), jnp.float32),
    ],
)
def _prop1_sc(y_hbm, e4_hbm, out_hbm, sidx, didx, yv, hist, hbuf, obuf, stg):
    cid = lax.axis_index("c")
    sid = lax.axis_index("s")
    wid = cid * NS + sid

    pltpu.sync_copy(e4_hbm.at[0, wid], sidx)
    pltpu.sync_copy(e4_hbm.at[1, wid], didx)
    pltpu.sync_copy(y_hbm, yv)
    _zero1_loop(hist, N // 16)

    def body(j, c):
        si = sidx[j]
        di = didx[j]
        vals = plsc.load_gather(yv, [si])
        plsc.addupdate_scatter(hist, [di], vals)
        return c

    lax.fori_loop(0, NV, body, 0)

    pltpu.sync_copy(hist, stg.at[sid])
    plsc.subcore_barrier()
    _hist_reduce_out(stg, hbuf, obuf, out_hbm, cid, sid)


def _mm_body(x_ref, w_ref, o_ref):
    o_ref[...] = jnp.dot(x_ref[...], w_ref[...],
                         preferred_element_type=jnp.float32)


def _mm(x, W1):
    return pl.pallas_call(
        _mm_body,
        out_shape=jax.ShapeDtypeStruct((N, H), jnp.float32),
    )(x, W1)


def _prep_body(degp_ref, xw_ref, dinv_ref, y1_ref):
    deg = degp_ref[pl.ds(0, N)] + degp_ref[pl.ds(N, N)] + 1.0
    dinv = lax.rsqrt(deg)
    dinv_ref[...] = dinv
    y1_ref[...] = xw_ref[...] * dinv[:, None]


def _prep(degp, xw):
    return pl.pallas_call(
        _prep_body,
        out_shape=[jax.ShapeDtypeStruct((N,), jnp.float32),
                   jax.ShapeDtypeStruct((N, H), jnp.float32)],
    )(degp, xw)


def _l2_body(accp_ref, y1_ref, dinv_ref, w2_ref, b1_ref, y2_ref):
    a = accp_ref[0] + accp_ref[1] + y1_ref[...]
    h = jnp.maximum(a * dinv_ref[...][:, None] + b1_ref[...], 0.0)
    hw = jnp.dot(h, w2_ref[...], preferred_element_type=jnp.float32)
    y2_ref[...] = hw[:, 0] * dinv_ref[...]


def _l2(accp, y1, dinv, W2, b1_row):
    return pl.pallas_call(
        _l2_body,
        out_shape=jax.ShapeDtypeStruct((N,), jnp.float32),
    )(accp, y1, dinv, W2, b1_row)


def _fin_body(acc2_ref, y2_ref, dinv_ref, b2_ref, out_ref):
    a = acc2_ref[pl.ds(0, N)] + acc2_ref[pl.ds(N, N)]
    out_ref[...] = ((a + y2_ref[...]) * dinv_ref[...] + b2_ref[0])[:, None]


def _fin(acc2, y2, dinv, b2):
    return pl.pallas_call(
        _fin_body,
        out_shape=jax.ShapeDtypeStruct((N, 1), jnp.float32),
    )(acc2, y2, dinv, b2)


def kernel(x, e, W1, b1, W2, b2):
    e3 = jnp.reshape(e, (2, NW, NCH, CP))
    e4 = jnp.reshape(e, (2, NW, NV, 16))

    degp = _deg_sc(e4)
    xw = _mm(x, W1)
    dinv, y1 = _prep(degp, xw)
    accp = _prop64_sc(y1, e3)
    y2 = _l2(accp, y1, dinv, W2, jnp.reshape(b1, (1, H)))
    acc2 = _prop1_sc(y2, e4)
    out = _fin(acc2, y2, dinv, b2)
    return out


# speedup vs baseline: 71.9338x; 1.0112x over previous
import functools

import jax
import jax.numpy as jnp
from jax import lax
from jax.experimental import pallas as pl
from jax.experimental.pallas import tpu as pltpu
from jax.experimental.pallas import tpu_sc as plsc

N = 10000
E = 320000
D = 128
H = 64

NC = 2
NS = 16
NW = NC * NS
EPT = E // NW
CP = 80
NCH = EPT // CP
NBUF = 5
WCH = 125
NV = EPT // 16
REG = 640

_MESH = plsc.VectorSubcoreMesh(core_axis_name="c", subcore_axis_name="s")
_NOTC = pltpu.CompilerParams(use_tc_tiling_on_sc=False,
                             needs_layout_passes=False)

_Z16 = functools.partial(jnp.zeros, (16,), jnp.float32)


def _zero2(buf, rows):
    def zb(r, c):
        for k in range(H // 16):
            buf[r, pl.ds(k * 16, 16)] = _Z16()
        return c
    lax.fori_loop(0, rows, zb, 0)


def _zero1_loop(buf, nvec):
    def zb(i, c):
        buf[pl.ds(i * 16, 16)] = _Z16()
        return c
    lax.fori_loop(0, nvec, zb, 0)


def _hist_reduce_out(stg, hbuf, obuf, out_hbm, cid, sid):
    def _region(r0, nv16):
        pltpu.sync_copy(stg.at[:, pl.ds(r0, nv16 * 16)],
                        hbuf.at[:, pl.ds(0, nv16 * 16)])

        def red(i, c):
            s = hbuf[0, pl.ds(i * 16, 16)]
            for k in range(1, NS):
                s = s + hbuf[k, pl.ds(i * 16, 16)]
            obuf[pl.ds(i * 16, 16)] = s
            return c
        lax.fori_loop(0, nv16, red, 0)
        pltpu.sync_copy(obuf.at[pl.ds(0, nv16 * 16)],
                        out_hbm.at[pl.ds(cid * N + r0, nv16 * 16)])

    @pl.when(sid < NS - 1)
    def _():
        _region(sid * REG, REG // 16)

    @pl.when(sid == NS - 1)
    def _():
        _region((NS - 1) * REG, (N - (NS - 1) * REG) // 16)


@functools.partial(
    pl.kernel,
    mesh=_MESH,
    compiler_params=_NOTC,
    out_type=jax.ShapeDtypeStruct((NC * N,), jnp.float32),
    scratch_types=[
        pltpu.VMEM((EPT,), jnp.int32),
        pltpu.VMEM((N,), jnp.float32),
        pltpu.VMEM((NS, REG), jnp.float32),
        pltpu.VMEM((REG,), jnp.float32),
        pltpu.VMEM_SHARED((NS, N), jnp.float32),
    ],
)
def _deg_sc(e5_hbm, out_hbm, didx, hist, hbuf, obuf, stg):
    cid = lax.axis_index("c")
    sid = lax.axis_index("s")
    wid = cid * NS + sid

    pltpu.sync_copy(e5_hbm.at[1, wid], didx)
    _zero1_loop(hist, N // 16)

    ones = jnp.full((16,), 1.0, jnp.float32)

    def body(g, c):
        for u in range(5):
            di = didx[pl.ds((g * 5 + u) * 16, 16)]
            plsc.addupdate_scatter(hist, [di], ones)
        return c

    lax.fori_loop(0, NV // 5, body, 0)

    pltpu.sync_copy(hist, stg.at[sid])
    plsc.subcore_barrier()
    _hist_reduce_out(stg, hbuf, obuf, out_hbm, cid, sid)


@functools.partial(
    pl.kernel,
    mesh=_MESH,
    compiler_params=_NOTC,
    out_type=jax.ShapeDtypeStruct((NC, N, H), jnp.float32),
    scratch_types=(
        [pltpu.VMEM((EPT,), jnp.int32),
         pltpu.VMEM((EPT,), jnp.int32)]
        + [pltpu.VMEM((WCH, H), jnp.float32) for _ in range(NBUF)]
        + [pltpu.VMEM_SHARED((N, H), jnp.float32)]
        + [pltpu.SemaphoreType.DMA for _ in range(2 * NBUF)]
    ),
)
def _prop64_sc(y_hbm, e5_hbm, out_hbm, sidx, didx, *rest):
    rows = rest[:NBUF]
    acc = rest[NBUF]
    sem_g = rest[NBUF + 1:2 * NBUF + 1]
    sem_s = rest[2 * NBUF + 1:]
    cid = lax.axis_index("c")
    sid = lax.axis_index("s")
    wid = cid * NS + sid
    nrt = N // NS

    pltpu.sync_copy(e5_hbm.at[0, wid], sidx)
    pltpu.sync_copy(e5_hbm.at[1, wid], didx)

    _zero2(rows[0], WCH)
    for k in range(nrt // WCH):
        pltpu.sync_copy(rows[0], acc.at[pl.ds(sid * nrt + k * WCH, WCH)])

    plsc.subcore_barrier()

    def body(g, carry):
        gd = []
        for b in range(NBUF):
            @pl.when(g > 0)
            def _(b=b):
                pltpu.make_async_copy(rows[b].at[pl.ds(0, CP)],
                                      acc.at[didx.at[pl.ds(0, CP)]],
                                      sem_s[b]).wait()
            gd.append(pltpu.async_copy(y_hbm.at[sidx.at[pl.ds((g * NBUF + b) * CP, CP)]],
                                       rows[b].at[pl.ds(0, CP)], sem_g[b]))
        for b in range(NBUF):
            gd[b].wait()
            pltpu.async_copy(rows[b].at[pl.ds(0, CP)],
                             acc.at[didx.at[pl.ds((g * NBUF + b) * CP, CP)]],
                             sem_s[b], add=True)
        return carry

    lax.fori_loop(0, NCH // NBUF, body, 0)
    for b in range(NBUF):
        pltpu.make_async_copy(rows[b].at[pl.ds(0, CP)],
                              acc.at[didx.at[pl.ds(0, CP)]], sem_s[b]).wait()

    plsc.subcore_barrier()

    wo = []
    for k in range(nrt // WCH):
        r0 = sid * nrt + k * WCH
        wo.append(pltpu.async_copy(acc.at[pl.ds(r0, WCH)], rows[k], sem_g[k]))
    wo2 = []
    for k in range(nrt // WCH):
        r0 = sid * nrt + k * WCH
        wo[k].wait()
        wo2.append(pltpu.async_copy(rows[k], out_hbm.at[cid, pl.ds(r0, WCH)],
                                    sem_s[k]))
    for d in wo2:
        d.wait()


@functools.partial(
    pl.kernel,
    mesh=_MESH,
    compiler_params=_NOTC,
    out_type=jax.ShapeDtypeStruct((NC * N,), jnp.float32),
    scratch_types=[
        pltpu.VMEM((EPT,), jnp.int32),
        pltpu.VMEM((EPT,), jnp.int32),
        pltpu.VMEM((N,), jnp.float32),
        pltpu.VMEM((N,), jnp.float32),
        pltpu.VMEM((NS, REG), jnp.float32),
        pltpu.VMEM((REG,), jnp.float32),
        pltpu.VMEM_SHARED((NS, N), jnp.float32),
    ],
)
def _prop1_sc(y_hbm, e5_hbm, out_hbm, sidx, didx, yv, hist, hbuf, obuf, stg):
    cid = lax.axis_index("c")
    sid = lax.axis_index("s")
    wid = cid * NS + sid

    pltpu.sync_copy(e5_hbm.at[0, wid], sidx)
    pltpu.sync_copy(e5_hbm.at[1, wid], didx)
    pltpu.sync_copy(y_hbm, yv)
    _zero1_loop(hist, N // 16)

    def body(g, c):
        for u in range(5):
            j16 = (g * 5 + u) * 16
            si = sidx[pl.ds(j16, 16)]
            di = didx[pl.ds(j16, 16)]
            vals = plsc.load_gather(yv, [si])
            plsc.addupdate_scatter(hist, [di], vals)
        return c

    lax.fori_loop(0, NV // 5, body, 0)

    pltpu.sync_copy(hist, stg.at[sid])
    plsc.subcore_barrier()
    _hist_reduce_out(stg, hbuf, obuf, out_hbm, cid, sid)


def _prep_body(x_ref, w_ref, degp_ref, dinv_ref, y1_ref):
    deg = degp_ref[pl.ds(0, N)] + degp_ref[pl.ds(N, N)] + 1.0
    dinv = lax.rsqrt(deg)
    dinv_ref[...] = dinv
    xw = jnp.dot(x_ref[...], w_ref[...], preferred_element_type=jnp.float32)
    y1_ref[...] = xw * dinv[:, None]


def _prep(x, W1, degp):
    return pl.pallas_call(
        _prep_body,
        out_shape=[jax.ShapeDtypeStruct((N,), jnp.float32),
                   jax.ShapeDtypeStruct((N, H), jnp.float32)],
    )(x, W1, degp)


def _l2_body(accp_ref, y1_ref, dinv_ref, w2_ref, b1_ref, y2_ref):
    a = accp_ref[0] + accp_ref[1] + y1_ref[...]
    h = jnp.maximum(a * dinv_ref[...][:, None] + b1_ref[...], 0.0)
    hw = jnp.dot(h, w2_ref[...], preferred_element_type=jnp.float32)
    y2_ref[...] = hw[:, 0] * dinv_ref[...]


def _l2(accp, y1, dinv, W2, b1_row):
    return pl.pallas_call(
        _l2_body,
        out_shape=jax.ShapeDtypeStruct((N,), jnp.float32),
    )(accp, y1, dinv, W2, b1_row)


def _fin_body(acc2_ref, y2_ref, dinv_ref, b2_ref, out_ref):
    a = acc2_ref[pl.ds(0, N)] + acc2_ref[pl.ds(N, N)]
    out_ref[...] = ((a + y2_ref[...]) * dinv_ref[...] + b2_ref[0])[:, None]


def _fin(acc2, y2, dinv, b2):
    return pl.pallas_call(
        _fin_body,
        out_shape=jax.ShapeDtypeStruct((N, 1), jnp.float32),
    )(acc2, y2, dinv, b2)


def kernel(x, e, W1, b1, W2, b2):
    e5 = jnp.reshape(e, (2, NW, EPT))

    degp = _deg_sc(e5)
    dinv, y1 = _prep(x, W1, degp)
    accp = _prop64_sc(y1, e5)
    y2 = _l2(accp, y1, dinv, W2, jnp.reshape(b1, (1, H)))
    acc2 = _prop1_sc(y2, e5)
    out = _fin(acc2, y2, dinv, b2)
    return out
